# Initial kernel scaffold; baseline (speedup 1.0000x reference)
import jax
import jax.numpy as jnp
from jax.experimental import pallas as pl


def kernel(logits, index):
    raise NotImplementedError("write your pallas kernel here")


# speedup vs baseline: 512.0744x; 512.0744x over previous
import functools

import jax
import jax.numpy as jnp
from jax import lax
from jax.experimental import pallas as pl
from jax.experimental.pallas import tpu as pltpu
from jax.experimental.pallas import tpu_sc as plsc

N = 12_800_000
G = 128
NC, NS, L = 2, 16, 16
NW = NC * NS
CHUNK = N // NW
BLK = 8_000
NBLK = CHUNK // BLK
VPB = BLK // L
NEG = -3.0e38

_mesh = plsc.VectorSubcoreMesh(core_axis_name="c", subcore_axis_name="s")
_params = pltpu.CompilerParams(needs_layout_passes=False)


def _wid():
    return lax.axis_index("s") * NC + lax.axis_index("c")


def _p1_body(x_hbm, i_hbm, pm_hbm, ps_hbm, xbuf, ibuf, accm, accs):
    wid = _wid()
    base = wid * CHUNK
    lane0 = lax.iota(jnp.int32, L) == 0

    for j in range(G // L):
        accm[pl.ds(j * L, L)] = jnp.full((L,), NEG, jnp.float32)
        accs[pl.ds(j * L, L)] = jnp.zeros((L,), jnp.float32)

    def merge(gvec, m_sc, s_sc):
        mold = plsc.load_gather(accm, [gvec])
        sold = plsc.load_gather(accs, [gvec])
        mnew = jnp.maximum(mold, m_sc)
        snew = sold * jnp.exp(mold - mnew) + s_sc * jnp.exp(m_sc - mnew)
        plsc.store_scatter(accm, [gvec], mnew, mask=lane0)
        plsc.store_scatter(accs, [gvec], snew, mask=lane0)

    def block_body(b, _):
        off = pl.multiple_of(base + b * BLK, 8)
        pltpu.sync_copy(x_hbm.at[pl.ds(off, BLK)], xbuf)
        pltpu.sync_copy(i_hbm.at[pl.ds(off, BLK)], ibuf)
        g0 = ibuf[pl.ds(0, L)][0]
        g1 = ibuf[pl.ds(BLK - L, L)][L - 1]

        def uniform():
            def mx(v, m):
                return jnp.maximum(m, xbuf[pl.ds(v * L, L)])
            mv = lax.fori_loop(0, VPB, mx, jnp.full((L,), NEG, jnp.float32))
            m_sc = jnp.max(mv)

            def sm(v, s):
                return s + jnp.exp(xbuf[pl.ds(v * L, L)] - m_sc)
            sv = lax.fori_loop(0, VPB, sm, jnp.zeros((L,), jnp.float32))
            merge(jnp.full((L,), g0, jnp.int32), m_sc, jnp.sum(sv))

        def mixed():
            def per_group(g, _):
                def mx(v, m):
                    xv = xbuf[pl.ds(v * L, L)]
                    iv = ibuf[pl.ds(v * L, L)]
                    return jnp.maximum(m, jnp.where(iv == g, xv, NEG))
                mv = lax.fori_loop(0, VPB, mx, jnp.full((L,), NEG, jnp.float32))
                m_sc = jnp.max(mv)

                def sm(v, s):
                    xv = xbuf[pl.ds(v * L, L)]
                    iv = ibuf[pl.ds(v * L, L)]
                    return s + jnp.where(iv == g, jnp.exp(xv - m_sc), 0.0)
                sv = lax.fori_loop(0, VPB, sm, jnp.zeros((L,), jnp.float32))
                merge(jnp.full((L,), g, jnp.int32), m_sc, jnp.sum(sv))
                return None

            lax.fori_loop(g0, g1 + 1, per_group, None)

        lax.cond(g0 == g1, uniform, mixed)
        return None

    lax.fori_loop(0, NBLK, block_body, None)
    pltpu.sync_copy(accm, pm_hbm.at[wid])
    pltpu.sync_copy(accs, ps_hbm.at[wid])


_pass1 = pl.kernel(
    _p1_body,
    out_type=(
        jax.ShapeDtypeStruct((NW, G), jnp.float32),
        jax.ShapeDtypeStruct((NW, G), jnp.float32),
    ),
    mesh=_mesh,
    compiler_params=_params,
    scratch_types=[
        pltpu.VMEM((BLK,), jnp.float32),
        pltpu.VMEM((BLK,), jnp.int32),
        pltpu.VMEM((G,), jnp.float32),
        pltpu.VMEM((G,), jnp.float32),
    ],
)


def _p2_body(x_hbm, i_hbm, c_hbm, o_hbm, xbuf, ibuf, obuf, cbuf):
    wid = _wid()
    base = wid * CHUNK
    pltpu.sync_copy(c_hbm, cbuf)

    def block_body(b, _):
        off = pl.multiple_of(base + b * BLK, 8)
        pltpu.sync_copy(x_hbm.at[pl.ds(off, BLK)], xbuf)
        pltpu.sync_copy(i_hbm.at[pl.ds(off, BLK)], ibuf)
        g0 = ibuf[pl.ds(0, L)][0]
        g1 = ibuf[pl.ds(BLK - L, L)][L - 1]

        def uniform():
            cv = plsc.load_gather(cbuf, [jnp.full((L,), g0, jnp.int32)])

            def body(v, _):
                obuf[pl.ds(v * L, L)] = xbuf[pl.ds(v * L, L)] - cv
                return None
            lax.fori_loop(0, VPB, body, None)

        def mixed():
            def body(v, _):
                iv = ibuf[pl.ds(v * L, L)]
                cv = plsc.load_gather(cbuf, [iv])
                obuf[pl.ds(v * L, L)] = xbuf[pl.ds(v * L, L)] - cv
                return None
            lax.fori_loop(0, VPB, body, None)

        lax.cond(g0 == g1, uniform, mixed)
        pltpu.sync_copy(obuf, o_hbm.at[pl.ds(off, BLK)])
        return None

    lax.fori_loop(0, NBLK, block_body, None)


_pass2 = pl.kernel(
    _p2_body,
    out_type=jax.ShapeDtypeStruct((N,), jnp.float32),
    mesh=_mesh,
    compiler_params=_params,
    scratch_types=[
        pltpu.VMEM((BLK,), jnp.float32),
        pltpu.VMEM((BLK,), jnp.int32),
        pltpu.VMEM((BLK,), jnp.float32),
        pltpu.VMEM((G,), jnp.float32),
    ],
)


def kernel(logits, index):
    pm, ps = _pass1(logits, index)
    gmax = jnp.max(pm, axis=0)
    gsum = jnp.sum(ps * jnp.exp(pm - gmax[None, :]), axis=0)
    c = gmax + jnp.log(gsum)
    return _pass2(logits, index, c)


# speedup vs baseline: 665.7425x; 1.3001x over previous
import functools

import jax
import jax.numpy as jnp
from jax import lax
from jax.experimental import pallas as pl
from jax.experimental.pallas import tpu as pltpu
from jax.experimental.pallas import tpu_sc as plsc

N = 12_800_000
G = 128
NC, NS, L = 2, 16, 16
NW = NC * NS
CHUNK = N // NW
BLK = 8_000
NBLK = CHUNK // BLK
VPB = BLK // L
NEG = -3.0e38

_mesh = plsc.VectorSubcoreMesh(core_axis_name="c", subcore_axis_name="s")
_params = pltpu.CompilerParams(needs_layout_passes=False)


def _wid():
    return lax.axis_index("s") * NC + lax.axis_index("c")


def _p1_body(x_hbm, i_hbm, pm_hbm, ps_hbm, xbuf, ibuf, accm, accs):
    wid = _wid()
    base = wid * CHUNK
    lane0 = lax.iota(jnp.int32, L) == 0

    for j in range(G // L):
        accm[pl.ds(j * L, L)] = jnp.full((L,), NEG, jnp.float32)
        accs[pl.ds(j * L, L)] = jnp.zeros((L,), jnp.float32)

    def merge(gvec, m_sc, s_sc):
        mold = plsc.load_gather(accm, [gvec])
        sold = plsc.load_gather(accs, [gvec])
        mnew = jnp.maximum(mold, m_sc)
        snew = sold * jnp.exp(mold - mnew) + s_sc * jnp.exp(m_sc - mnew)
        plsc.store_scatter(accm, [gvec], mnew, mask=lane0)
        plsc.store_scatter(accs, [gvec], snew, mask=lane0)

    def block_body(b, _):
        off = pl.multiple_of(base + b * BLK, 8)
        pltpu.sync_copy(x_hbm.at[pl.ds(off, BLK)], xbuf)
        pltpu.sync_copy(i_hbm.at[pl.ds(off, BLK)], ibuf)
        g0 = ibuf[pl.ds(0, L)][0]
        g1 = ibuf[pl.ds(BLK - L, L)][L - 1]

        def uniform():
            def mx(v, m):
                return jnp.maximum(m, xbuf[pl.ds(v * L, L)])
            mv = lax.fori_loop(0, VPB, mx, jnp.full((L,), NEG, jnp.float32), unroll=8)
            m_sc = jnp.max(mv)

            def sm(v, s):
                return s + jnp.exp(xbuf[pl.ds(v * L, L)] - m_sc)
            sv = lax.fori_loop(0, VPB, sm, jnp.zeros((L,), jnp.float32), unroll=8)
            merge(jnp.full((L,), g0, jnp.int32), m_sc, jnp.sum(sv))

        def mixed():
            def per_group(g, _):
                def mx(v, m):
                    xv = xbuf[pl.ds(v * L, L)]
                    iv = ibuf[pl.ds(v * L, L)]
                    return jnp.maximum(m, jnp.where(iv == g, xv, NEG))
                mv = lax.fori_loop(0, VPB, mx, jnp.full((L,), NEG, jnp.float32), unroll=8)
                m_sc = jnp.max(mv)

                def sm(v, s):
                    xv = xbuf[pl.ds(v * L, L)]
                    iv = ibuf[pl.ds(v * L, L)]
                    return s + jnp.where(iv == g, jnp.exp(xv - m_sc), 0.0)
                sv = lax.fori_loop(0, VPB, sm, jnp.zeros((L,), jnp.float32), unroll=8)
                merge(jnp.full((L,), g, jnp.int32), m_sc, jnp.sum(sv))
                return None

            lax.fori_loop(g0, g1 + 1, per_group, None)

        lax.cond(g0 == g1, uniform, mixed)
        return None

    lax.fori_loop(0, NBLK, block_body, None)
    pltpu.sync_copy(accm, pm_hbm.at[wid])
    pltpu.sync_copy(accs, ps_hbm.at[wid])


_pass1 = pl.kernel(
    _p1_body,
    out_type=(
        jax.ShapeDtypeStruct((NW, G), jnp.float32),
        jax.ShapeDtypeStruct((NW, G), jnp.float32),
    ),
    mesh=_mesh,
    compiler_params=_params,
    scratch_types=[
        pltpu.VMEM((BLK,), jnp.float32),
        pltpu.VMEM((BLK,), jnp.int32),
        pltpu.VMEM((G,), jnp.float32),
        pltpu.VMEM((G,), jnp.float32),
    ],
)


def _p2_body(x_hbm, i_hbm, c_hbm, o_hbm, xbuf, ibuf, obuf, cbuf):
    wid = _wid()
    base = wid * CHUNK
    pltpu.sync_copy(c_hbm, cbuf)

    def block_body(b, _):
        off = pl.multiple_of(base + b * BLK, 8)
        pltpu.sync_copy(x_hbm.at[pl.ds(off, BLK)], xbuf)
        pltpu.sync_copy(i_hbm.at[pl.ds(off, BLK)], ibuf)
        g0 = ibuf[pl.ds(0, L)][0]
        g1 = ibuf[pl.ds(BLK - L, L)][L - 1]

        def uniform():
            cv = plsc.load_gather(cbuf, [jnp.full((L,), g0, jnp.int32)])

            def body(v, _):
                obuf[pl.ds(v * L, L)] = xbuf[pl.ds(v * L, L)] - cv
                return None
            lax.fori_loop(0, VPB, body, None, unroll=8)

        def mixed():
            def body(v, _):
                iv = ibuf[pl.ds(v * L, L)]
                cv = plsc.load_gather(cbuf, [iv])
                obuf[pl.ds(v * L, L)] = xbuf[pl.ds(v * L, L)] - cv
                return None
            lax.fori_loop(0, VPB, body, None, unroll=8)

        lax.cond(g0 == g1, uniform, mixed)
        pltpu.sync_copy(obuf, o_hbm.at[pl.ds(off, BLK)])
        return None

    lax.fori_loop(0, NBLK, block_body, None)


_pass2 = pl.kernel(
    _p2_body,
    out_type=jax.ShapeDtypeStruct((N,), jnp.float32),
    mesh=_mesh,
    compiler_params=_params,
    scratch_types=[
        pltpu.VMEM((BLK,), jnp.float32),
        pltpu.VMEM((BLK,), jnp.int32),
        pltpu.VMEM((BLK,), jnp.float32),
        pltpu.VMEM((G,), jnp.float32),
    ],
)


def kernel(logits, index):
    pm, ps = _pass1(logits, index)
    gmax = jnp.max(pm, axis=0)
    gsum = jnp.sum(ps * jnp.exp(pm - gmax[None, :]), axis=0)
    c = gmax + jnp.log(gsum)
    return _pass2(logits, index, c)


# speedup vs baseline: 1276.8975x; 1.9180x over previous
import jax
import jax.numpy as jnp
from jax import lax
from jax.experimental import pallas as pl
from jax.experimental.pallas import tpu as pltpu
from jax.experimental.pallas import tpu_sc as plsc

N = 12_800_000
G = 128
NC, NS, L = 2, 16, 16
NW = NC * NS
CHUNK = N // NW
BLK = 8_000
NBLK = CHUNK // BLK
HALF = NBLK // 2
VPB = BLK // L
UNROLL = 8
NEG = -3.0e38

_mesh = plsc.VectorSubcoreMesh(core_axis_name="c", subcore_axis_name="s")
_params = pltpu.CompilerParams(needs_layout_passes=False)


def _wid():
    return lax.axis_index("s") * NC + lax.axis_index("c")


def _p1_body(x_hbm, i_hbm, pm_hbm, ps_hbm,
             xb0, xb1, ib0, ib1, accm, accs, sx0, sx1, si0, si1):
    wid = _wid()
    base = wid * CHUNK
    lane0 = lax.iota(jnp.int32, L) == 0
    xb, ib, sx, si = [xb0, xb1], [ib0, ib1], [sx0, sx1], [si0, si1]

    for j in range(G // L):
        accm[pl.ds(j * L, L)] = jnp.full((L,), NEG, jnp.float32)
        accs[pl.ds(j * L, L)] = jnp.zeros((L,), jnp.float32)

    def start(j, off):
        pltpu.make_async_copy(x_hbm.at[pl.ds(off, BLK)], xb[j], sx[j]).start()
        pltpu.make_async_copy(i_hbm.at[pl.ds(off, BLK)], ib[j], si[j]).start()

    def wait(j):
        pltpu.make_async_copy(x_hbm.at[pl.ds(0, BLK)], xb[j], sx[j]).wait()
        pltpu.make_async_copy(i_hbm.at[pl.ds(0, BLK)], ib[j], si[j]).wait()

    def merge(gvec, m_sc, s_sc):
        mold = plsc.load_gather(accm, [gvec])
        sold = plsc.load_gather(accs, [gvec])
        mnew = jnp.maximum(mold, m_sc)
        snew = sold * jnp.exp(mold - mnew) + s_sc * jnp.exp(m_sc - mnew)
        plsc.store_scatter(accm, [gvec], mnew, mask=lane0)
        plsc.store_scatter(accs, [gvec], snew, mask=lane0)

    def compute(j):
        xbuf, ibuf = xb[j], ib[j]
        g0 = ibuf[pl.ds(0, L)][0]
        g1 = ibuf[pl.ds(BLK - L, L)][L - 1]

        def uniform():
            def mx(v, m):
                return jnp.maximum(m, xbuf[pl.ds(v * L, L)])
            mv = lax.fori_loop(0, VPB, mx, jnp.full((L,), NEG, jnp.float32),
                               unroll=UNROLL)
            m_sc = jnp.max(mv)

            def sm(v, s):
                return s + jnp.exp(xbuf[pl.ds(v * L, L)] - m_sc)
            sv = lax.fori_loop(0, VPB, sm, jnp.zeros((L,), jnp.float32),
                               unroll=UNROLL)
            merge(jnp.full((L,), g0, jnp.int32), m_sc, jnp.sum(sv))

        def mixed():
            def per_group(g, _):
                def mx(v, m):
                    xv = xbuf[pl.ds(v * L, L)]
                    iv = ibuf[pl.ds(v * L, L)]
                    return jnp.maximum(m, jnp.where(iv == g, xv, NEG))
                mv = lax.fori_loop(0, VPB, mx, jnp.full((L,), NEG, jnp.float32),
                                   unroll=UNROLL)
                m_sc = jnp.max(mv)

                def sm(v, s):
                    xv = xbuf[pl.ds(v * L, L)]
                    iv = ibuf[pl.ds(v * L, L)]
                    return s + jnp.where(iv == g, jnp.exp(xv - m_sc), 0.0)
                sv = lax.fori_loop(0, VPB, sm, jnp.zeros((L,), jnp.float32),
                                   unroll=UNROLL)
                merge(jnp.full((L,), g, jnp.int32), m_sc, jnp.sum(sv))
                return None

            lax.fori_loop(g0, g1 + 1, per_group, None)

        lax.cond(g0 == g1, uniform, mixed)

    start(0, base)

    def super_body(i, _):
        b0 = 2 * i
        start(1, base + (b0 + 1) * BLK)
        wait(0)
        compute(0)
        off2 = lax.select(b0 + 2 < NBLK, base + (b0 + 2) * BLK, base)
        start(0, off2)
        wait(1)
        compute(1)
        return None

    lax.fori_loop(0, HALF, super_body, None)
    wait(0)
    pltpu.sync_copy(accm, pm_hbm.at[wid])
    pltpu.sync_copy(accs, ps_hbm.at[wid])


_pass1 = pl.kernel(
    _p1_body,
    out_type=(
        jax.ShapeDtypeStruct((NW, G), jnp.float32),
        jax.ShapeDtypeStruct((NW, G), jnp.float32),
    ),
    mesh=_mesh,
    compiler_params=_params,
    scratch_types=[
        pltpu.VMEM((BLK,), jnp.float32),
        pltpu.VMEM((BLK,), jnp.float32),
        pltpu.VMEM((BLK,), jnp.int32),
        pltpu.VMEM((BLK,), jnp.int32),
        pltpu.VMEM((G,), jnp.float32),
        pltpu.VMEM((G,), jnp.float32),
        pltpu.SemaphoreType.DMA,
        pltpu.SemaphoreType.DMA,
        pltpu.SemaphoreType.DMA,
        pltpu.SemaphoreType.DMA,
    ],
)


def _p2_body(x_hbm, i_hbm, c_hbm, o_hbm,
             xb0, xb1, ib0, ib1, ob0, ob1, cbuf,
             sx0, sx1, si0, si1, so0, so1):
    wid = _wid()
    base = wid * CHUNK
    xb, ib, ob = [xb0, xb1], [ib0, ib1], [ob0, ob1]
    sx, si, so = [sx0, sx1], [si0, si1], [so0, so1]
    pltpu.sync_copy(c_hbm, cbuf)

    def start(j, off):
        pltpu.make_async_copy(x_hbm.at[pl.ds(off, BLK)], xb[j], sx[j]).start()
        pltpu.make_async_copy(i_hbm.at[pl.ds(off, BLK)], ib[j], si[j]).start()

    def wait(j):
        pltpu.make_async_copy(x_hbm.at[pl.ds(0, BLK)], xb[j], sx[j]).wait()
        pltpu.make_async_copy(i_hbm.at[pl.ds(0, BLK)], ib[j], si[j]).wait()

    def start_out(j, off):
        pltpu.make_async_copy(ob[j], o_hbm.at[pl.ds(off, BLK)], so[j]).start()

    def wait_out(j):
        pltpu.make_async_copy(ob[j], o_hbm.at[pl.ds(0, BLK)], so[j]).wait()

    def compute(j):
        xbuf, ibuf, obuf = xb[j], ib[j], ob[j]
        g0 = ibuf[pl.ds(0, L)][0]
        g1 = ibuf[pl.ds(BLK - L, L)][L - 1]

        def uniform():
            cv = plsc.load_gather(cbuf, [jnp.full((L,), g0, jnp.int32)])

            def body(v, _):
                obuf[pl.ds(v * L, L)] = xbuf[pl.ds(v * L, L)] - cv
                return None
            lax.fori_loop(0, VPB, body, None, unroll=UNROLL)

        def mixed():
            def body(v, _):
                iv = ibuf[pl.ds(v * L, L)]
                cv = plsc.load_gather(cbuf, [iv])
                obuf[pl.ds(v * L, L)] = xbuf[pl.ds(v * L, L)] - cv
                return None
            lax.fori_loop(0, VPB, body, None, unroll=UNROLL)

        lax.cond(g0 == g1, uniform, mixed)

    start(0, base)

    def super_body(i, _):
        b0 = 2 * i
        start(1, base + (b0 + 1) * BLK)
        wait(0)
        lax.cond(i > 0, lambda: wait_out(0), lambda: None)
        compute(0)
        start_out(0, base + b0 * BLK)
        off2 = lax.select(b0 + 2 < NBLK, base + (b0 + 2) * BLK, base)
        start(0, off2)
        wait(1)
        lax.cond(i > 0, lambda: wait_out(1), lambda: None)
        compute(1)
        start_out(1, base + (b0 + 1) * BLK)
        return None

    lax.fori_loop(0, HALF, super_body, None)
    wait(0)
    wait_out(0)
    wait_out(1)


_pass2 = pl.kernel(
    _p2_body,
    out_type=jax.ShapeDtypeStruct((N,), jnp.float32),
    mesh=_mesh,
    compiler_params=_params,
    scratch_types=[
        pltpu.VMEM((BLK,), jnp.float32),
        pltpu.VMEM((BLK,), jnp.float32),
        pltpu.VMEM((BLK,), jnp.int32),
        pltpu.VMEM((BLK,), jnp.int32),
        pltpu.VMEM((BLK,), jnp.float32),
        pltpu.VMEM((BLK,), jnp.float32),
        pltpu.VMEM((G,), jnp.float32),
        pltpu.SemaphoreType.DMA,
        pltpu.SemaphoreType.DMA,
        pltpu.SemaphoreType.DMA,
        pltpu.SemaphoreType.DMA,
        pltpu.SemaphoreType.DMA,
        pltpu.SemaphoreType.DMA,
    ],
)


def kernel(logits, index):
    pm, ps = _pass1(logits, index)
    gmax = jnp.max(pm, axis=0)
    gsum = jnp.sum(ps * jnp.exp(pm - gmax[None, :]), axis=0)
    c = gmax + jnp.log(gsum)
    return _pass2(logits, index, c)
